# baseline (speedup 1.0000x reference)
import functools

import jax
import jax.numpy as jnp
from jax.experimental import pallas as pl
from jax.experimental.pallas import tpu as pltpu

B = 4096
D = 1024
E = 4
K = 2
HG = 512
HC = 512
OUT = 3
LANES = 128
BB = 512
NSTEPS = B // BB


def _layernorm(x, gamma, beta, eps=1e-5):
    mu = jnp.mean(x, axis=-1, keepdims=True)
    var = jnp.mean(x * x, axis=-1, keepdims=True) - mu * mu
    return (x - mu) * jax.lax.rsqrt(var + eps) * gamma + beta


def _gelu(x):
    return 0.5 * x * (1.0 + jax.lax.erf(x * 0.7071067811865476))


def _ec_copy(ec_hbm, ec_buf, sem, step, slot):
    return pltpu.make_async_copy(
        ec_hbm.at[:, pl.ds(step * BB, BB), :], ec_buf.at[slot], sem.at[slot])


def _fused_kernel(gc_ref, ec_hbm, wg1_hbm, bg1_ref, g1g_ref, g1b_ref,
                  wg2_hbm, bg2_ref, wc1_hbm, bc1_ref, c1g_ref, c1b_ref,
                  wc2_hbm, bc2_ref, logits_out_ref, probs_out_ref,
                  ec_buf, sem, wg1_ref, wg2_ref, wc1_ref, wc2_ref, wsem):
    i = pl.program_id(0)
    slot = jax.lax.rem(i, 2)
    nxt = jax.lax.rem(i + 1, 2)

    @pl.when(i == 0)
    def _start_first():
        _ec_copy(ec_hbm, ec_buf, sem, 0, 0).start()
        for k, (src, dst) in enumerate([(wg1_hbm, wg1_ref), (wg2_hbm, wg2_ref),
                                        (wc1_hbm, wc1_ref), (wc2_hbm, wc2_ref)]):
            pltpu.make_async_copy(src, dst, wsem.at[k]).start()
        for k, (src, dst) in enumerate([(wg1_hbm, wg1_ref), (wg2_hbm, wg2_ref),
                                        (wc1_hbm, wc1_ref), (wc2_hbm, wc2_ref)]):
            pltpu.make_async_copy(src, dst, wsem.at[k]).wait()

    @pl.when(i + 1 < NSTEPS)
    def _start_next():
        _ec_copy(ec_hbm, ec_buf, sem, i + 1, nxt).start()

    h = jnp.dot(gc_ref[...], wg1_ref[...],
                preferred_element_type=jnp.float32) + bg1_ref[...]
    h = _layernorm(h, g1g_ref[...], g1b_ref[...])
    h = _gelu(h)
    logits = jnp.dot(h, wg2_ref[...],
                     preferred_element_type=jnp.float32) + bg2_ref[...]
    l4 = logits[:, :E]
    m = jnp.max(l4, axis=-1, keepdims=True)
    p = jnp.exp(l4 - m)
    p = p / jnp.sum(p, axis=-1, keepdims=True)
    probs_out_ref[...] = p
    idx = jax.lax.broadcasted_iota(jnp.int32, p.shape, 1)
    m1 = jnp.max(p, axis=-1, keepdims=True)
    i1 = jnp.min(jnp.where(p == m1, idx, E), axis=-1, keepdims=True)
    p_wo = jnp.where(idx == i1, -1.0, p)
    m2 = jnp.max(p_wo, axis=-1, keepdims=True)
    i2 = jnp.min(jnp.where(p_wo == m2, idx, E), axis=-1, keepdims=True)
    inv = 1.0 / (m1 + m2)
    gates = jnp.where((idx == i1) | (idx == i2), p, 0.0) * inv
    _ec_copy(ec_hbm, ec_buf, sem, i, slot).wait()
    cp = gates[:, 0:1] * ec_buf[slot, 0]
    for e in range(1, E):
        cp = cp + gates[:, e:e + 1] * ec_buf[slot, e]
    hc = jnp.dot(cp, wc1_ref[...],
                 preferred_element_type=jnp.float32) + bc1_ref[...]
    hc = _layernorm(hc, c1g_ref[...], c1b_ref[...])
    hc = _gelu(hc)
    out = jnp.dot(hc, wc2_ref[...],
                  preferred_element_type=jnp.float32) + bc2_ref[...]
    logits_out_ref[...] = out[:, :OUT]


@functools.partial(jax.jit, static_argnames=("interpret",))
def kernel(gating_cls, expert_cls, Wg1, bg1, g1_gamma, g1_beta, Wg2, bg2,
           Wc1, bc1, c1_gamma, c1_beta, Wc2, bc2, interpret=False):
    Wg2p = jnp.zeros((HG, LANES), jnp.float32).at[:, :E].set(Wg2)
    bg2p = jnp.full((1, LANES), -1e30, jnp.float32).at[0, :E].set(bg2)
    Wc2p = jnp.zeros((HC, LANES), jnp.float32).at[:, :OUT].set(Wc2)
    bc2p = jnp.zeros((1, LANES), jnp.float32).at[0, :OUT].set(bc2)
    row = lambda v: v.reshape(1, -1)

    out_logits, out_probs = pl.pallas_call(
        _fused_kernel,
        grid=(NSTEPS,),
        in_specs=[
            pl.BlockSpec((BB, D), lambda i: (i, 0)),
            pl.BlockSpec(memory_space=pl.ANY),
            pl.BlockSpec(memory_space=pl.ANY),
            pl.BlockSpec((1, HG), lambda i: (0, 0)),
            pl.BlockSpec((1, HG), lambda i: (0, 0)),
            pl.BlockSpec((1, HG), lambda i: (0, 0)),
            pl.BlockSpec(memory_space=pl.ANY),
            pl.BlockSpec((1, LANES), lambda i: (0, 0)),
            pl.BlockSpec(memory_space=pl.ANY),
            pl.BlockSpec((1, HC), lambda i: (0, 0)),
            pl.BlockSpec((1, HC), lambda i: (0, 0)),
            pl.BlockSpec((1, HC), lambda i: (0, 0)),
            pl.BlockSpec(memory_space=pl.ANY),
            pl.BlockSpec((1, LANES), lambda i: (0, 0)),
        ],
        out_specs=[
            pl.BlockSpec((BB, OUT), lambda i: (i, 0)),
            pl.BlockSpec((BB, E), lambda i: (i, 0)),
        ],
        out_shape=[
            jax.ShapeDtypeStruct((B, OUT), jnp.float32),
            jax.ShapeDtypeStruct((B, E), jnp.float32),
        ],
        scratch_shapes=[
            pltpu.VMEM((2, E, BB, D), jnp.float32),
            pltpu.SemaphoreType.DMA((2,)),
            pltpu.VMEM((D, HG), jnp.float32),
            pltpu.VMEM((HG, LANES), jnp.float32),
            pltpu.VMEM((D, HC), jnp.float32),
            pltpu.VMEM((HC, LANES), jnp.float32),
            pltpu.SemaphoreType.DMA((4,)),
        ],
        interpret=interpret,
    )(gating_cls, expert_cls, Wg1, row(bg1), row(g1_gamma), row(g1_beta),
      Wg2p, bg2p, Wc1, row(bc1), row(c1_gamma), row(c1_beta), Wc2p, bc2p)
    return out_logits, out_probs


# speedup vs baseline: 1.0689x; 1.0689x over previous
import functools

import jax
import jax.numpy as jnp
from jax.experimental import pallas as pl
from jax.experimental.pallas import tpu as pltpu

B = 4096
D = 1024
E = 4
K = 2
HG = 512
HC = 512
OUT = 3
LANES = 128
BB = 1024
NSTEPS = B // BB


def _layernorm(x, gamma, beta, eps=1e-5):
    mu = jnp.mean(x, axis=-1, keepdims=True)
    var = jnp.mean(x * x, axis=-1, keepdims=True) - mu * mu
    return (x - mu) * jax.lax.rsqrt(var + eps) * gamma + beta


def _gelu(x):
    return 0.5 * x * (1.0 + jax.lax.erf(x * 0.7071067811865476))


def _ec_copy(ec_hbm, ec_buf, sem, step, slot):
    return pltpu.make_async_copy(
        ec_hbm.at[:, pl.ds(step * BB, BB), :], ec_buf.at[slot], sem.at[slot])


def _fused_kernel(gc_ref, ec_hbm, wg1_ref, bg1_ref, g1g_ref, g1b_ref,
                  wg2_ref, bg2_ref, wc1_ref, bc1_ref, c1g_ref, c1b_ref,
                  wc2_ref, bc2_ref, logits_out_ref, probs_out_ref,
                  ec_buf, sem):
    i = pl.program_id(0)
    slot = jax.lax.rem(i, 2)
    nxt = jax.lax.rem(i + 1, 2)

    @pl.when(i == 0)
    def _start_first():
        _ec_copy(ec_hbm, ec_buf, sem, 0, 0).start()

    @pl.when(i + 1 < NSTEPS)
    def _start_next():
        _ec_copy(ec_hbm, ec_buf, sem, i + 1, nxt).start()

    h = jnp.dot(gc_ref[...], wg1_ref[...],
                preferred_element_type=jnp.float32) + bg1_ref[...]
    h = _layernorm(h, g1g_ref[...], g1b_ref[...])
    h = _gelu(h)
    logits = jnp.dot(h, wg2_ref[...],
                     preferred_element_type=jnp.float32) + bg2_ref[...]
    l4 = logits[:, :E]
    m = jnp.max(l4, axis=-1, keepdims=True)
    p = jnp.exp(l4 - m)
    p = p / jnp.sum(p, axis=-1, keepdims=True)
    probs_out_ref[...] = p
    idx = jax.lax.broadcasted_iota(jnp.int32, p.shape, 1)
    m1 = jnp.max(p, axis=-1, keepdims=True)
    i1 = jnp.min(jnp.where(p == m1, idx, E), axis=-1, keepdims=True)
    p_wo = jnp.where(idx == i1, -1.0, p)
    m2 = jnp.max(p_wo, axis=-1, keepdims=True)
    i2 = jnp.min(jnp.where(p_wo == m2, idx, E), axis=-1, keepdims=True)
    inv = 1.0 / (m1 + m2)
    gates = jnp.where((idx == i1) | (idx == i2), p, 0.0) * inv
    _ec_copy(ec_hbm, ec_buf, sem, i, slot).wait()
    cp = gates[:, 0:1] * ec_buf[slot, 0]
    for e in range(1, E):
        cp = cp + gates[:, e:e + 1] * ec_buf[slot, e]
    hc = jnp.dot(cp, wc1_ref[...],
                 preferred_element_type=jnp.float32) + bc1_ref[...]
    hc = _layernorm(hc, c1g_ref[...], c1b_ref[...])
    hc = _gelu(hc)
    out = jnp.dot(hc, wc2_ref[...],
                  preferred_element_type=jnp.float32) + bc2_ref[...]
    logits_out_ref[...] = out[:, :OUT]


@functools.partial(jax.jit, static_argnames=("interpret",))
def kernel(gating_cls, expert_cls, Wg1, bg1, g1_gamma, g1_beta, Wg2, bg2,
           Wc1, bc1, c1_gamma, c1_beta, Wc2, bc2, interpret=False):
    Wg2p = jnp.zeros((HG, LANES), jnp.float32).at[:, :E].set(Wg2)
    bg2p = jnp.full((1, LANES), -1e30, jnp.float32).at[0, :E].set(bg2)
    Wc2p = jnp.zeros((HC, LANES), jnp.float32).at[:, :OUT].set(Wc2)
    bc2p = jnp.zeros((1, LANES), jnp.float32).at[0, :OUT].set(bc2)
    row = lambda v: v.reshape(1, -1)

    out_logits, out_probs = pl.pallas_call(
        _fused_kernel,
        grid=(NSTEPS,),
        in_specs=[
            pl.BlockSpec((BB, D), lambda i: (i, 0)),
            pl.BlockSpec(memory_space=pl.ANY),
            pl.BlockSpec((D, HG), lambda i: (0, 0)),
            pl.BlockSpec((1, HG), lambda i: (0, 0)),
            pl.BlockSpec((1, HG), lambda i: (0, 0)),
            pl.BlockSpec((1, HG), lambda i: (0, 0)),
            pl.BlockSpec((HG, LANES), lambda i: (0, 0)),
            pl.BlockSpec((1, LANES), lambda i: (0, 0)),
            pl.BlockSpec((D, HC), lambda i: (0, 0)),
            pl.BlockSpec((1, HC), lambda i: (0, 0)),
            pl.BlockSpec((1, HC), lambda i: (0, 0)),
            pl.BlockSpec((1, HC), lambda i: (0, 0)),
            pl.BlockSpec((HC, LANES), lambda i: (0, 0)),
            pl.BlockSpec((1, LANES), lambda i: (0, 0)),
        ],
        out_specs=[
            pl.BlockSpec((BB, OUT), lambda i: (i, 0)),
            pl.BlockSpec((BB, E), lambda i: (i, 0)),
        ],
        out_shape=[
            jax.ShapeDtypeStruct((B, OUT), jnp.float32),
            jax.ShapeDtypeStruct((B, E), jnp.float32),
        ],
        scratch_shapes=[
            pltpu.VMEM((2, E, BB, D), jnp.float32),
            pltpu.SemaphoreType.DMA((2,)),
        ],
        interpret=interpret,
    )(gating_cls, expert_cls, Wg1, row(bg1), row(g1_gamma), row(g1_beta),
      Wg2p, bg2p, Wc1, row(bc1), row(c1_gamma), row(c1_beta), Wc2p, bc2p)
    return out_logits, out_probs


# speedup vs baseline: 1.0752x; 1.0059x over previous
import jax
import jax.numpy as jnp
from jax.experimental import pallas as pl
from jax.experimental.pallas import tpu as pltpu

B = 4096
D = 1024
E = 4
K = 2
HG = 512
HC = 512
OUT = 3
LANES = 128
BB = 1024
NSTEPS = B // BB


def _layernorm(x, gamma, beta, eps=1e-5):
    mu = jnp.mean(x, axis=-1, keepdims=True)
    var = jnp.mean(x * x, axis=-1, keepdims=True) - mu * mu
    return (x - mu) * jax.lax.rsqrt(var + eps) * gamma + beta


def _gelu(x):
    return 0.5 * x * (1.0 + jax.lax.erf(x * 0.7071067811865476))


def _ec_copy(ec_hbm, ec_buf, sem, step, slot):
    return pltpu.make_async_copy(
        ec_hbm.at[:, pl.ds(step * BB, BB), :], ec_buf.at[slot], sem.at[slot])


def _fused_kernel(gc_ref, ec_hbm, wg1_ref, bg1_ref, g1g_ref, g1b_ref,
                  wg2_ref, bg2_ref, wc1_ref, bc1_ref, c1g_ref, c1b_ref,
                  wc2_ref, bc2_ref, logits_out_ref, probs_out_ref,
                  ec_buf, sem):
    i = pl.program_id(0)
    slot = jax.lax.rem(i, 2)
    nxt = jax.lax.rem(i + 1, 2)

    @pl.when(i == 0)
    def _start_first():
        _ec_copy(ec_hbm, ec_buf, sem, 0, 0).start()

    @pl.when(i + 1 < NSTEPS)
    def _start_next():
        _ec_copy(ec_hbm, ec_buf, sem, i + 1, nxt).start()

    h = jnp.dot(gc_ref[...], wg1_ref[...],
                preferred_element_type=jnp.float32) + bg1_ref[...]
    h = _layernorm(h, g1g_ref[...], g1b_ref[...])
    h = _gelu(h)
    logits = jnp.dot(h, wg2_ref[...],
                     preferred_element_type=jnp.float32) + bg2_ref[...]
    l4 = logits[:, :E]
    m = jnp.max(l4, axis=-1, keepdims=True)
    p = jnp.exp(l4 - m)
    p = p / jnp.sum(p, axis=-1, keepdims=True)
    probs_out_ref[...] = p
    idx = jax.lax.broadcasted_iota(jnp.int32, p.shape, 1)
    m1 = jnp.max(p, axis=-1, keepdims=True)
    i1 = jnp.min(jnp.where(p == m1, idx, E), axis=-1, keepdims=True)
    p_wo = jnp.where(idx == i1, -1.0, p)
    m2 = jnp.max(p_wo, axis=-1, keepdims=True)
    i2 = jnp.min(jnp.where(p_wo == m2, idx, E), axis=-1, keepdims=True)
    inv = 1.0 / (m1 + m2)
    gates = jnp.where((idx == i1) | (idx == i2), p, 0.0) * inv
    _ec_copy(ec_hbm, ec_buf, sem, i, slot).wait()
    cp = gates[:, 0:1] * ec_buf[slot, 0]
    for e in range(1, E):
        cp = cp + gates[:, e:e + 1] * ec_buf[slot, e]
    hc = jnp.dot(cp, wc1_ref[...],
                 preferred_element_type=jnp.float32) + bc1_ref[...]
    hc = _layernorm(hc, c1g_ref[...], c1b_ref[...])
    hc = _gelu(hc)
    out = jnp.dot(hc, wc2_ref[...],
                  preferred_element_type=jnp.float32) + bc2_ref[...]
    logits_out_ref[...] = out[:, :OUT]


@jax.jit
def kernel(gating_cls, expert_cls, Wg1, bg1, g1_gamma, g1_beta, Wg2, bg2,
           Wc1, bc1, c1_gamma, c1_beta, Wc2, bc2):
    Wg2p = jnp.zeros((HG, LANES), jnp.float32).at[:, :E].set(Wg2)
    bg2p = jnp.full((1, LANES), -1e30, jnp.float32).at[0, :E].set(bg2)
    Wc2p = jnp.zeros((HC, LANES), jnp.float32).at[:, :OUT].set(Wc2)
    bc2p = jnp.zeros((1, LANES), jnp.float32).at[0, :OUT].set(bc2)
    row = lambda v: v.reshape(1, -1)

    out_logits, out_probs = pl.pallas_call(
        _fused_kernel,
        grid=(NSTEPS,),
        in_specs=[
            pl.BlockSpec((BB, D), lambda i: (i, 0)),
            pl.BlockSpec(memory_space=pl.ANY),
            pl.BlockSpec((D, HG), lambda i: (0, 0)),
            pl.BlockSpec((1, HG), lambda i: (0, 0)),
            pl.BlockSpec((1, HG), lambda i: (0, 0)),
            pl.BlockSpec((1, HG), lambda i: (0, 0)),
            pl.BlockSpec((HG, LANES), lambda i: (0, 0)),
            pl.BlockSpec((1, LANES), lambda i: (0, 0)),
            pl.BlockSpec((D, HC), lambda i: (0, 0)),
            pl.BlockSpec((1, HC), lambda i: (0, 0)),
            pl.BlockSpec((1, HC), lambda i: (0, 0)),
            pl.BlockSpec((1, HC), lambda i: (0, 0)),
            pl.BlockSpec((HC, LANES), lambda i: (0, 0)),
            pl.BlockSpec((1, LANES), lambda i: (0, 0)),
        ],
        out_specs=[
            pl.BlockSpec((BB, OUT), lambda i: (i, 0)),
            pl.BlockSpec((BB, E), lambda i: (i, 0)),
        ],
        out_shape=[
            jax.ShapeDtypeStruct((B, OUT), jnp.float32),
            jax.ShapeDtypeStruct((B, E), jnp.float32),
        ],
        scratch_shapes=[
            pltpu.VMEM((2, E, BB, D), jnp.float32),
            pltpu.SemaphoreType.DMA((2,)),
        ],
    )(gating_cls, expert_cls, Wg1, row(bg1), row(g1_gamma), row(g1_beta),
      Wg2p, bg2p, Wc1, row(bc1), row(c1_gamma), row(c1_beta), Wc2p, bc2p)
    return out_logits, out_probs
